# baseline (speedup 1.0000x reference)
import functools

import jax
import jax.numpy as jnp
from jax import lax
from jax.experimental import pallas as pl
from jax.experimental.pallas import tpu as pltpu
from jax.experimental.pallas import tpu_sc as plsc

_INFO = plsc.get_sparse_core_info()
_NC = _INFO.num_cores
_NS = _INFO.num_subcores
_NW = _NC * _NS
_CHUNKS = 4
_NBUF = 4
_LAG = 3
_BASE = 10000.0
_BS = 1024


@functools.lru_cache(maxsize=None)
def _build_sc_gather(nb: int, s: int, dim: int):
    w_per_b = _NW // nb
    b_per_w = s // w_per_b
    rows = b_per_w // _CHUNKS
    assert rows * _CHUNKS * w_per_b == s and rows % 8 == 0
    mesh = plsc.VectorSubcoreMesh(core_axis_name="c", subcore_axis_name="s")

    @functools.partial(
        pl.kernel,
        mesh=mesh,
        out_type=jax.ShapeDtypeStruct((nb, s, dim), jnp.float32),
        scratch_types=[
            pltpu.VMEM((b_per_w,), jnp.int32),
            pltpu.VMEM((_NBUF, rows, dim), jnp.float32),
            pltpu.SemaphoreType.DMA((_NBUF,)),
            pltpu.SemaphoreType.DMA((_NBUF,)),
        ],
    )
    def gather(tbl_hbm, idx_hbm, out_hbm, idx_v, bufs, gsems, wsems):
        wid = lax.axis_index("s") * _NC + lax.axis_index("c")
        bi = wid // w_per_b
        off = (wid % w_per_b) * b_per_w
        pltpu.sync_copy(idx_hbm.at[bi, pl.ds(off, b_per_w)], idx_v)

        gh = {}
        wh = {}
        waited = set()

        def start_gather(t):
            b = t % _NBUF
            gh[t] = pltpu.async_copy(
                tbl_hbm.at[idx_v.at[pl.ds(t * rows, rows)]], bufs.at[b],
                gsems.at[b])

        for t in range(_LAG):
            start_gather(t)
        for t in range(_CHUNKS):
            nxt = t + _LAG
            if nxt < _CHUNKS:
                prev = nxt - _NBUF
                if prev >= 0:
                    wh[prev].wait()
                    waited.add(prev)
                start_gather(nxt)
            b = t % _NBUF
            gh[t].wait()
            wh[t] = pltpu.async_copy(
                bufs.at[b], out_hbm.at[bi, pl.ds(off + t * rows, rows)],
                wsems.at[b])
        for t in range(_CHUNKS):
            if t not in waited:
                wh[t].wait()

    return gather


@functools.lru_cache(maxsize=None)
def _build_tc_sin(nb: int, s: int, dim: int):
    def body(p_ref, f_ref, o_ref):
        i = pl.program_id(0)
        j = pl.program_id(1)
        p = p_ref[i, pl.ds(j * _BS, _BS)]
        ang = p.astype(jnp.float32)[:, None] * f_ref[0, :][None, :]
        o_ref[0] = jnp.sin(ang)

    return pl.pallas_call(
        body,
        grid=(nb, s // _BS),
        in_specs=[
            pl.BlockSpec((nb, s), lambda i, j: (0, 0)),
            pl.BlockSpec((1, dim), lambda i, j: (0, 0)),
        ],
        out_specs=pl.BlockSpec((1, _BS, dim), lambda i, j: (i, j, 0)),
        out_shape=jax.ShapeDtypeStruct((nb, s, dim), jnp.float32),
    )


def kernel(x, position_ids, cos_cached, sin_cached):
    nb, s = position_ids.shape
    dim = cos_cached.shape[-1]
    pid = position_ids.astype(jnp.int32)
    inv_freq = 1.0 / (_BASE ** (jnp.arange(0, dim, 2, dtype=jnp.float32) / dim))
    inv2 = jnp.concatenate([inv_freq, inv_freq]).reshape(1, dim)
    cos = _build_sc_gather(nb, s, dim)(cos_cached, pid)
    sin = _build_tc_sin(nb, s, dim)(pid, inv2)
    return cos.astype(x.dtype), sin.astype(x.dtype)


# speedup vs baseline: 1.3322x; 1.3322x over previous
import functools

import jax
import jax.numpy as jnp
from jax import lax
from jax.experimental import pallas as pl
from jax.experimental.pallas import tpu as pltpu
from jax.experimental.pallas import tpu_sc as plsc

_INFO = plsc.get_sparse_core_info()
_NC = _INFO.num_cores
_NS = _INFO.num_subcores
_NW = _NC * _NS
_CHUNKS = 8
_NBUF = 6
_LAG = 5


@functools.lru_cache(maxsize=None)
def _build_gather(nb: int, s: int, dim: int):
    w_per_b = _NW // nb
    b_per_w = s // w_per_b
    rows = b_per_w // _CHUNKS
    assert rows * _CHUNKS * w_per_b == s and rows % 8 == 0
    n_tasks = 2 * _CHUNKS
    mesh = plsc.VectorSubcoreMesh(core_axis_name="c", subcore_axis_name="s")

    @functools.partial(
        pl.kernel,
        mesh=mesh,
        out_type=(
            jax.ShapeDtypeStruct((nb, s, dim), jnp.float32),
            jax.ShapeDtypeStruct((nb, s, dim), jnp.float32),
        ),
        scratch_types=[
            pltpu.VMEM((b_per_w,), jnp.int32),
            pltpu.VMEM((_NBUF, rows, dim), jnp.float32),
            pltpu.SemaphoreType.DMA((_NBUF,)),
            pltpu.SemaphoreType.DMA((_NBUF,)),
        ],
    )
    def gather(cos_hbm, sin_hbm, idx_hbm, cos_out, sin_out,
               idx_v, bufs, gsems, wsems):
        wid = lax.axis_index("s") * _NC + lax.axis_index("c")
        bi = wid // w_per_b
        off = (wid % w_per_b) * b_per_w
        pltpu.sync_copy(idx_hbm.at[bi, pl.ds(off, b_per_w)], idx_v)

        def task(t):
            tbl = cos_hbm if t < _CHUNKS else sin_hbm
            out = cos_out if t < _CHUNKS else sin_out
            return tbl, out, t % _CHUNKS

        gh = {}
        wh = {}
        waited = set()

        def start_gather(t):
            tbl, _, c = task(t)
            b = t % _NBUF
            gh[t] = pltpu.async_copy(
                tbl.at[idx_v.at[pl.ds(c * rows, rows)]], bufs.at[b],
                gsems.at[b])

        for t in range(_LAG):
            start_gather(t)
        for t in range(n_tasks):
            nxt = t + _LAG
            if nxt < n_tasks:
                prev = nxt - _NBUF
                if prev >= 0:
                    wh[prev].wait()
                    waited.add(prev)
                start_gather(nxt)
            _, out, c = task(t)
            b = t % _NBUF
            gh[t].wait()
            wh[t] = pltpu.async_copy(
                bufs.at[b], out.at[bi, pl.ds(off + c * rows, rows)],
                wsems.at[b])
        for t in range(n_tasks):
            if t not in waited:
                wh[t].wait()

    return gather


def kernel(x, position_ids, cos_cached, sin_cached):
    nb, s = position_ids.shape
    dim = cos_cached.shape[-1]
    cos, sin = _build_gather(nb, s, dim)(
        cos_cached, sin_cached, position_ids.astype(jnp.int32))
    return cos.astype(x.dtype), sin.astype(x.dtype)


# speedup vs baseline: 1.3544x; 1.0166x over previous
import functools

import jax
import jax.numpy as jnp
from jax import lax
from jax.experimental import pallas as pl
from jax.experimental.pallas import tpu as pltpu
from jax.experimental.pallas import tpu_sc as plsc

_INFO = plsc.get_sparse_core_info()
_NC = _INFO.num_cores
_NS = _INFO.num_subcores
_NW = _NC * _NS
_CHUNKS = 4
_NBUF = 4
_LAG = 3


@functools.lru_cache(maxsize=None)
def _build_gather(nb: int, s: int, dim: int):
    w_per_b = _NW // nb
    b_per_w = s // w_per_b
    rows = b_per_w // _CHUNKS
    assert rows * _CHUNKS * w_per_b == s and rows % 8 == 0
    n_tasks = 2 * _CHUNKS
    mesh = plsc.VectorSubcoreMesh(core_axis_name="c", subcore_axis_name="s")

    @functools.partial(
        pl.kernel,
        mesh=mesh,
        out_type=(
            jax.ShapeDtypeStruct((nb, s, dim), jnp.float32),
            jax.ShapeDtypeStruct((nb, s, dim), jnp.float32),
        ),
        scratch_types=[
            pltpu.VMEM((b_per_w,), jnp.int32),
            pltpu.VMEM((_NBUF, rows, dim), jnp.float32),
            pltpu.SemaphoreType.DMA((_NBUF,)),
            pltpu.SemaphoreType.DMA((_NBUF,)),
        ],
    )
    def gather(cos_hbm, sin_hbm, idx_hbm, cos_out, sin_out,
               idx_v, bufs, gsems, wsems):
        wid = lax.axis_index("s") * _NC + lax.axis_index("c")
        bi = wid // w_per_b
        off = (wid % w_per_b) * b_per_w
        pltpu.sync_copy(idx_hbm.at[bi, pl.ds(off, b_per_w)], idx_v)

        def task(t):
            tbl = cos_hbm if t < _CHUNKS else sin_hbm
            out = cos_out if t < _CHUNKS else sin_out
            return tbl, out, t % _CHUNKS

        gh = {}
        wh = {}
        waited = set()

        def start_gather(t):
            tbl, _, c = task(t)
            b = t % _NBUF
            gh[t] = pltpu.async_copy(
                tbl.at[idx_v.at[pl.ds(c * rows, rows)]], bufs.at[b],
                gsems.at[b])

        for t in range(_LAG):
            start_gather(t)
        for t in range(n_tasks):
            nxt = t + _LAG
            if nxt < n_tasks:
                prev = nxt - _NBUF
                if prev >= 0:
                    wh[prev].wait()
                    waited.add(prev)
                start_gather(nxt)
            _, out, c = task(t)
            b = t % _NBUF
            gh[t].wait()
            wh[t] = pltpu.async_copy(
                bufs.at[b], out.at[bi, pl.ds(off + c * rows, rows)],
                wsems.at[b])
        for t in range(n_tasks):
            if t not in waited:
                wh[t].wait()

    return gather


def kernel(x, position_ids, cos_cached, sin_cached):
    nb, s = position_ids.shape
    dim = cos_cached.shape[-1]
    cos, sin = _build_gather(nb, s, dim)(
        cos_cached, sin_cached, position_ids.astype(jnp.int32))
    return cos.astype(x.dtype), sin.astype(x.dtype)


# speedup vs baseline: 1.5641x; 1.1549x over previous
import functools

import numpy as np

import jax
import jax.numpy as jnp
from jax import lax
from jax.experimental import pallas as pl
from jax.experimental.pallas import tpu as pltpu
from jax.experimental.pallas import tpu_sc as plsc

_INFO = plsc.get_sparse_core_info()
_NC = _INFO.num_cores
_NS = _INFO.num_subcores
_NW = _NC * _NS
_CHUNKS = 4
_NBUF = 4
_LAG = 3
_BASE = 10000.0
_BS = 1024

_SIN_COEFFS = (6.2831826, -41.341423, 81.59619,
               -76.58015, 41.205597, -12.271582)


@functools.lru_cache(maxsize=None)
def _build_sc_gather(nb: int, s: int, dim: int):
    w_per_b = _NW // nb
    b_per_w = s // w_per_b
    rows = b_per_w // _CHUNKS
    assert rows * _CHUNKS * w_per_b == s and rows % 8 == 0
    mesh = plsc.VectorSubcoreMesh(core_axis_name="c", subcore_axis_name="s")

    @functools.partial(
        pl.kernel,
        mesh=mesh,
        out_type=jax.ShapeDtypeStruct((nb, s, dim), jnp.float32),
        scratch_types=[
            pltpu.VMEM((b_per_w,), jnp.int32),
            pltpu.VMEM((_NBUF, rows, dim), jnp.float32),
            pltpu.SemaphoreType.DMA((_NBUF,)),
            pltpu.SemaphoreType.DMA((_NBUF,)),
        ],
    )
    def gather(tbl_hbm, idx_hbm, out_hbm, idx_v, bufs, gsems, wsems):
        wid = lax.axis_index("s") * _NC + lax.axis_index("c")
        bi = wid // w_per_b
        off = (wid % w_per_b) * b_per_w
        pltpu.sync_copy(idx_hbm.at[bi, pl.ds(off, b_per_w)], idx_v)

        gh = {}
        wh = {}
        waited = set()

        def start_gather(t):
            b = t % _NBUF
            gh[t] = pltpu.async_copy(
                tbl_hbm.at[idx_v.at[pl.ds(t * rows, rows)]], bufs.at[b],
                gsems.at[b])

        for t in range(_LAG):
            start_gather(t)
        for t in range(_CHUNKS):
            nxt = t + _LAG
            if nxt < _CHUNKS:
                prev = nxt - _NBUF
                if prev >= 0:
                    wh[prev].wait()
                    waited.add(prev)
                start_gather(nxt)
            b = t % _NBUF
            gh[t].wait()
            wh[t] = pltpu.async_copy(
                bufs.at[b], out_hbm.at[bi, pl.ds(off + t * rows, rows)],
                wsems.at[b])
        for t in range(_CHUNKS):
            if t not in waited:
                wh[t].wait()

    return gather


@functools.lru_cache(maxsize=None)
def _build_tc_sin(nb: int, s: int, dim: int):
    def body(p_ref, st_ref, o_ref):
        i = pl.program_id(0)
        j = pl.program_id(1)
        p = p_ref[i, pl.ds(j * _BS, _BS)].astype(jnp.uint32)
        phase = p[:, None] * st_ref[0, :][None, :]
        xf = phase.astype(jnp.int32).astype(jnp.float32) * jnp.float32(2.0**-32)
        x2 = xf * xf
        acc = jnp.float32(_SIN_COEFFS[-1])
        for c in _SIN_COEFFS[-2::-1]:
            acc = acc * x2 + jnp.float32(c)
        o_ref[0] = acc * xf

    return pl.pallas_call(
        body,
        grid=(nb, s // _BS),
        in_specs=[
            pl.BlockSpec((nb, s), lambda i, j: (0, 0)),
            pl.BlockSpec((1, dim), lambda i, j: (0, 0)),
        ],
        out_specs=pl.BlockSpec((1, _BS, dim), lambda i, j: (i, j, 0)),
        out_shape=jax.ShapeDtypeStruct((nb, s, dim), jnp.float32),
    )


def _phase_steps(dim: int) -> jnp.ndarray:
    inv_freq = 1.0 / (_BASE ** (np.arange(0, dim, 2, dtype=np.float64) / dim))
    frac = (inv_freq / (2.0 * np.pi)) % 1.0
    steps = (np.round(frac * 2.0**32).astype(np.uint64) % (1 << 32)).astype(
        np.uint32)
    return jnp.asarray(np.concatenate([steps, steps]).reshape(1, dim))


def kernel(x, position_ids, cos_cached, sin_cached):
    nb, s = position_ids.shape
    dim = cos_cached.shape[-1]
    pid = position_ids.astype(jnp.int32)
    cos = _build_sc_gather(nb, s, dim)(cos_cached, pid)
    sin = _build_tc_sin(nb, s, dim)(pid, _phase_steps(dim))
    return cos.astype(x.dtype), sin.astype(x.dtype)


# speedup vs baseline: 1.5647x; 1.0004x over previous
import functools

import numpy as np

import jax
import jax.numpy as jnp
from jax import lax
from jax.experimental import pallas as pl
from jax.experimental.pallas import tpu as pltpu
from jax.experimental.pallas import tpu_sc as plsc

_INFO = plsc.get_sparse_core_info()
_NC = _INFO.num_cores
_NS = _INFO.num_subcores
_NW = _NC * _NS
_CHUNKS = 2
_NBUF = 2
_LAG = 1
_BASE = 10000.0
_BS = 1024

_SIN_COEFFS = (6.2831826, -41.341423, 81.59619,
               -76.58015, 41.205597, -12.271582)


@functools.lru_cache(maxsize=None)
def _build_sc_gather(nb: int, s: int, dim: int):
    w_per_b = _NW // nb
    b_per_w = s // w_per_b
    rows = b_per_w // _CHUNKS
    assert rows * _CHUNKS * w_per_b == s and rows % 8 == 0
    mesh = plsc.VectorSubcoreMesh(core_axis_name="c", subcore_axis_name="s")

    @functools.partial(
        pl.kernel,
        mesh=mesh,
        out_type=jax.ShapeDtypeStruct((nb, s, dim), jnp.float32),
        scratch_types=[
            pltpu.VMEM((b_per_w,), jnp.int32),
            pltpu.VMEM((_NBUF, rows, dim), jnp.float32),
            pltpu.SemaphoreType.DMA((_NBUF,)),
            pltpu.SemaphoreType.DMA((_NBUF,)),
        ],
    )
    def gather(tbl_hbm, idx_hbm, out_hbm, idx_v, bufs, gsems, wsems):
        wid = lax.axis_index("s") * _NC + lax.axis_index("c")
        bi = wid // w_per_b
        off = (wid % w_per_b) * b_per_w
        pltpu.sync_copy(idx_hbm.at[bi, pl.ds(off, b_per_w)], idx_v)

        gh = {}
        wh = {}
        waited = set()

        def start_gather(t):
            b = t % _NBUF
            gh[t] = pltpu.async_copy(
                tbl_hbm.at[idx_v.at[pl.ds(t * rows, rows)]], bufs.at[b],
                gsems.at[b])

        for t in range(_LAG):
            start_gather(t)
        for t in range(_CHUNKS):
            nxt = t + _LAG
            if nxt < _CHUNKS:
                prev = nxt - _NBUF
                if prev >= 0:
                    wh[prev].wait()
                    waited.add(prev)
                start_gather(nxt)
            b = t % _NBUF
            gh[t].wait()
            wh[t] = pltpu.async_copy(
                bufs.at[b], out_hbm.at[bi, pl.ds(off + t * rows, rows)],
                wsems.at[b])
        for t in range(_CHUNKS):
            if t not in waited:
                wh[t].wait()

    return gather


@functools.lru_cache(maxsize=None)
def _build_tc_sin(nb: int, s: int, dim: int):
    def body(p_ref, st_ref, o_ref):
        i = pl.program_id(0)
        j = pl.program_id(1)
        p = p_ref[i, pl.ds(j * _BS, _BS)].astype(jnp.uint32)
        phase = p[:, None] * st_ref[0, :][None, :]
        xf = phase.astype(jnp.int32).astype(jnp.float32) * jnp.float32(2.0**-32)
        x2 = xf * xf
        acc = jnp.float32(_SIN_COEFFS[-1])
        for c in _SIN_COEFFS[-2::-1]:
            acc = acc * x2 + jnp.float32(c)
        o_ref[0] = acc * xf

    return pl.pallas_call(
        body,
        grid=(nb, s // _BS),
        in_specs=[
            pl.BlockSpec((nb, s), lambda i, j: (0, 0)),
            pl.BlockSpec((1, dim), lambda i, j: (0, 0)),
        ],
        out_specs=pl.BlockSpec((1, _BS, dim), lambda i, j: (i, j, 0)),
        out_shape=jax.ShapeDtypeStruct((nb, s, dim), jnp.float32),
    )


def _phase_steps(dim: int) -> jnp.ndarray:
    inv_freq = 1.0 / (_BASE ** (np.arange(0, dim, 2, dtype=np.float64) / dim))
    frac = (inv_freq / (2.0 * np.pi)) % 1.0
    steps = (np.round(frac * 2.0**32).astype(np.uint64) % (1 << 32)).astype(
        np.uint32)
    return jnp.asarray(np.concatenate([steps, steps]).reshape(1, dim))


def kernel(x, position_ids, cos_cached, sin_cached):
    nb, s = position_ids.shape
    dim = cos_cached.shape[-1]
    pid = position_ids.astype(jnp.int32)
    cos = _build_sc_gather(nb, s, dim)(cos_cached, pid)
    sin = _build_tc_sin(nb, s, dim)(pid, _phase_steps(dim))
    return cos.astype(x.dtype), sin.astype(x.dtype)
